# Initial kernel scaffold; baseline (speedup 1.0000x reference)
import functools

import jax
import jax.numpy as jnp
from jax import lax
from jax.experimental import pallas as pl
from jax.experimental.pallas import tpu as pltpu
from jax.experimental.pallas import tpu_sc as plsc

N_NODES = 10000
NP = 10240
N_EDGES = 320000
CHUNK = 128
NW = 32
K = 79
E_PAD = NW * K * CHUNK
SUB_ROWS = NP // 16


def _make_agg(D):
    mesh = plsc.VectorSubcoreMesh(core_axis_name="c", subcore_axis_name="s")

    @functools.partial(
        pl.kernel,
        mesh=mesh,
        out_type=jax.ShapeDtypeStruct((2, NP, D), jnp.float32),
        scratch_types=[
            pltpu.VMEM((K, CHUNK), jnp.int32),
            pltpu.VMEM((K, CHUNK), jnp.int32),
            pltpu.VMEM((CHUNK, D), jnp.float32),
            pltpu.VMEM_SHARED((NP, D), jnp.float32),
        ],
    )
    def agg(y_hbm, src_hbm, dst_hbm, zero_hbm, out_hbm,
            src_v, dst_v, rows_v, accum_sh):
        c = lax.axis_index("c")
        s = lax.axis_index("s")
        wid = s * 2 + c
        base = s * SUB_ROWS
        pltpu.sync_copy(zero_hbm, accum_sh.at[pl.ds(base, SUB_ROWS)])
        plsc.subcore_barrier()
        pltpu.sync_copy(src_hbm.at[wid], src_v)
        pltpu.sync_copy(dst_hbm.at[wid], dst_v)

        def body(j, carry):
            pltpu.sync_copy(y_hbm.at[src_v.at[j]], rows_v)
            pltpu.sync_copy(rows_v, accum_sh.at[dst_v.at[j]], add=True)
            return carry

        lax.fori_loop(0, K, body, 0)
        plsc.subcore_barrier()
        pltpu.sync_copy(accum_sh.at[pl.ds(base, SUB_ROWS)],
                        out_hbm.at[c, pl.ds(base, SUB_ROWS)])

    return agg


_AGG = {D: _make_agg(D) for D in (64, 128, 192)}


def _agg_pass(y, src_p, dst_p, zeros_cache):
    D = y.shape[1]
    z = zeros_cache[D]
    out = _AGG[D](y, src_p, dst_p, z)
    return out[0, :N_NODES] + out[1, :N_NODES]


def _bn(x, g, b, eps=1e-5):
    mu = jnp.mean(x, axis=0)
    var = jnp.var(x, axis=0)
    return (x - mu) / jnp.sqrt(var + eps) * g + b


def _leaky(x):
    return jnp.where(x >= 0, x, 0.01 * x)


def _som_dists(x, W):
    d2 = (jnp.sum(x * x, axis=1, keepdims=True) - 2.0 * (x @ W.T)
          + jnp.sum(W * W, axis=1))
    return jnp.sqrt(jnp.maximum(d2, 1e-12))


def _pools(h, batch, n_graphs=64):
    s = jax.ops.segment_sum(h, batch, num_segments=n_graphs)
    cnt = jax.ops.segment_sum(jnp.ones((h.shape[0], 1), h.dtype), batch,
                              num_segments=n_graphs)
    avg = s / jnp.maximum(cnt, 1.0)
    mx = jax.ops.segment_max(h, batch, num_segments=n_graphs)
    return jnp.concatenate([avg, s, mx], axis=1)


def kernel(x, edge_index, batch, params):
    p = params
    src = edge_index[0]
    dst = edge_index[1]
    pad = E_PAD - N_EDGES
    src_p = jnp.concatenate(
        [src, jnp.zeros((pad,), jnp.int32)]).reshape(NW, K, CHUNK)
    dst_p = jnp.concatenate(
        [dst, jnp.full((pad,), N_NODES, jnp.int32)]).reshape(NW, K, CHUNK)
    zeros_cache = {D: jnp.zeros((SUB_ROWS, D), jnp.float32)
                   for D in (64, 128, 192)}

    y0 = x @ p['conv1_Wrel'].T
    agg0 = _agg_pass(y0, src_p, dst_p, zeros_cache)
    x1 = _bn(_leaky(agg0 + p['conv1_b'] + x @ p['conv1_Wroot'].T),
             p['norm1_g'], p['norm1_b'])

    so1 = _som_dists(x1, p['som1_W'])
    p1 = so1 @ p['oc1_Wrel'].T
    pay2 = jnp.concatenate([x1, p1], axis=1)
    agg2 = _agg_pass(pay2, src_p, dst_p, zeros_cache)
    x2 = _bn(_leaky(agg2[:, :64] @ p['conv2_Wrel'].T + p['conv2_b']
                    + x1 @ p['conv2_Wroot'].T), p['norm2_g'], p['norm2_b'])
    h1 = _bn(_leaky(agg2[:, 64:] + p['oc1_b'] + so1 @ p['oc1_Wroot'].T),
             p['on1_g'], p['on1_b'])

    so2 = _som_dists(x2, p['som2_W'])
    p2 = so2 @ p['oc2_Wrel'].T
    pay3 = jnp.concatenate([x2, p2], axis=1)
    agg3 = _agg_pass(pay3, src_p, dst_p, zeros_cache)
    x3 = _bn(_leaky(agg3[:, :128] @ p['conv3_Wrel'].T + p['conv3_b']
                    + x2 @ p['conv3_Wroot'].T), p['norm3_g'], p['norm3_b'])
    h2 = _bn(_leaky(agg3[:, 128:] + p['oc2_b'] + so2 @ p['oc2_Wroot'].T),
             p['on2_g'], p['on2_b'])

    so3 = _som_dists(x3, p['som3_W'])
    p3 = so3 @ p['oc3_Wrel'].T
    agg4 = _agg_pass(p3, src_p, dst_p, zeros_cache)
    h3 = _bn(_leaky(agg4 + p['oc3_b'] + so3 @ p['oc3_Wroot'].T),
             p['on3_g'], p['on3_b'])

    h_conv = jnp.concatenate([x1, x2, x3], axis=1)
    h_GNN = _pools(h_conv, batch)
    gnn_out = jax.nn.log_softmax(h_GNN @ p['lin_GNN_W'].T + p['lin_GNN_b'],
                                 axis=1)
    som_out_conv = jnp.concatenate([h1, h2, h3], axis=1)
    hp = _pools(som_out_conv, batch)
    h = jax.nn.log_softmax(hp @ p['lin_out_W'].T + p['lin_out_b'], axis=1)
    return (h, h_conv, gnn_out)


# speedup vs baseline: 3.1194x; 3.1194x over previous
import functools

import jax
import jax.numpy as jnp
from jax import lax
from jax.experimental import pallas as pl
from jax.experimental.pallas import tpu as pltpu
from jax.experimental.pallas import tpu_sc as plsc

N_NODES = 10000
NP = 10240
N_EDGES = 320000
CHUNK = 128
NW = 32
K = 79
E_PAD = NW * K * CHUNK
SUB_ROWS = NP // 16


def _make_agg(D):
    mesh = plsc.VectorSubcoreMesh(core_axis_name="c", subcore_axis_name="s")

    @functools.partial(
        pl.kernel,
        mesh=mesh,
        out_type=jax.ShapeDtypeStruct((2, NP, D), jnp.float32),
        scratch_types=[
            pltpu.VMEM((K, CHUNK), jnp.int32),
            pltpu.VMEM((K, CHUNK), jnp.int32),
            pltpu.VMEM((CHUNK, D), jnp.float32),
            pltpu.VMEM_SHARED((NP, D), jnp.float32),
        ],
        compiler_params=pltpu.CompilerParams(use_tc_tiling_on_sc=False),
    )
    def agg(y_hbm, src_hbm, dst_hbm, zero_hbm, out_hbm,
            src_v, dst_v, rows_v, accum_sh):
        c = lax.axis_index("c")
        s = lax.axis_index("s")
        wid = s * 2 + c
        base = s * SUB_ROWS
        pltpu.sync_copy(zero_hbm, accum_sh.at[pl.ds(base, SUB_ROWS)])
        plsc.subcore_barrier()
        pltpu.sync_copy(src_hbm.at[wid], src_v)
        pltpu.sync_copy(dst_hbm.at[wid], dst_v)

        def body(j, carry):
            pltpu.sync_copy(y_hbm.at[src_v.at[j]], rows_v)
            pltpu.sync_copy(rows_v, accum_sh.at[dst_v.at[j]], add=True)
            return carry

        lax.fori_loop(0, K, body, 0)
        plsc.subcore_barrier()
        pltpu.sync_copy(accum_sh.at[pl.ds(base, SUB_ROWS)],
                        out_hbm.at[c, pl.ds(base, SUB_ROWS)])

    return agg


_AGG = {D: _make_agg(D) for D in (64, 112, 128)}


def _agg_pass(y, src_p, dst_p, zeros_cache):
    D = y.shape[1]
    z = zeros_cache[D]
    out = _AGG[D](y, src_p, dst_p, z)
    return out[0, :N_NODES] + out[1, :N_NODES]


def _bn(x, g, b, eps=1e-5):
    mu = jnp.mean(x, axis=0)
    var = jnp.var(x, axis=0)
    return (x - mu) / jnp.sqrt(var + eps) * g + b


def _leaky(x):
    return jnp.where(x >= 0, x, 0.01 * x)


def _som_dists(x, W):
    d2 = (jnp.sum(x * x, axis=1, keepdims=True) - 2.0 * (x @ W.T)
          + jnp.sum(W * W, axis=1))
    return jnp.sqrt(jnp.maximum(d2, 1e-12))


def _pools(h, batch, n_graphs=64):
    s = jax.ops.segment_sum(h, batch, num_segments=n_graphs)
    cnt = jax.ops.segment_sum(jnp.ones((h.shape[0], 1), h.dtype), batch,
                              num_segments=n_graphs)
    avg = s / jnp.maximum(cnt, 1.0)
    mx = jax.ops.segment_max(h, batch, num_segments=n_graphs)
    return jnp.concatenate([avg, s, mx], axis=1)


def kernel(x, edge_index, batch, params):
    p = params
    src = edge_index[0]
    dst = edge_index[1]
    pad = E_PAD - N_EDGES
    src_p = jnp.concatenate(
        [src, jnp.zeros((pad,), jnp.int32)]).reshape(NW, K, CHUNK)
    dst_p = jnp.concatenate(
        [dst, jnp.full((pad,), N_NODES, jnp.int32)]).reshape(NW, K, CHUNK)
    zeros_cache = {D: jnp.zeros((SUB_ROWS, D), jnp.float32)
                   for D in (64, 112, 128)}
    pad100 = jnp.zeros((N_NODES, 12), jnp.float32)

    agg1 = _agg_pass(x, src_p, dst_p, zeros_cache)
    x1 = _bn(_leaky(agg1 @ p['conv1_Wrel'].T + p['conv1_b']
                    + x @ p['conv1_Wroot'].T), p['norm1_g'], p['norm1_b'])

    agg2 = _agg_pass(x1, src_p, dst_p, zeros_cache)
    x2 = _bn(_leaky(agg2 @ p['conv2_Wrel'].T + p['conv2_b']
                    + x1 @ p['conv2_Wroot'].T), p['norm2_g'], p['norm2_b'])

    agg3 = _agg_pass(x2, src_p, dst_p, zeros_cache)
    x3 = _bn(_leaky(agg3 @ p['conv3_Wrel'].T + p['conv3_b']
                    + x2 @ p['conv3_Wroot'].T), p['norm3_g'], p['norm3_b'])

    so1 = _som_dists(x1, p['som1_W'])
    so2 = _som_dists(x2, p['som2_W'])
    so3 = _som_dists(x3, p['som3_W'])
    aggs1 = _agg_pass(jnp.concatenate([so1, pad100], axis=1),
                      src_p, dst_p, zeros_cache)[:, :100]
    aggs2 = _agg_pass(jnp.concatenate([so2, pad100], axis=1),
                      src_p, dst_p, zeros_cache)[:, :100]
    aggs3 = _agg_pass(jnp.concatenate([so3, pad100], axis=1),
                      src_p, dst_p, zeros_cache)[:, :100]
    h1 = _bn(_leaky(aggs1 @ p['oc1_Wrel'].T + p['oc1_b']
                    + so1 @ p['oc1_Wroot'].T), p['on1_g'], p['on1_b'])
    h2 = _bn(_leaky(aggs2 @ p['oc2_Wrel'].T + p['oc2_b']
                    + so2 @ p['oc2_Wroot'].T), p['on2_g'], p['on2_b'])
    h3 = _bn(_leaky(aggs3 @ p['oc3_Wrel'].T + p['oc3_b']
                    + so3 @ p['oc3_Wroot'].T), p['on3_g'], p['on3_b'])

    h_conv = jnp.concatenate([x1, x2, x3], axis=1)
    h_GNN = _pools(h_conv, batch)
    gnn_out = jax.nn.log_softmax(h_GNN @ p['lin_GNN_W'].T + p['lin_GNN_b'],
                                 axis=1)
    som_out_conv = jnp.concatenate([h1, h2, h3], axis=1)
    hp = _pools(som_out_conv, batch)
    h = jax.nn.log_softmax(hp @ p['lin_out_W'].T + p['lin_out_b'], axis=1)
    return (h, h_conv, gnn_out)
